# Initial kernel scaffold; baseline (speedup 1.0000x reference)
import jax
import jax.numpy as jnp
from jax.experimental import pallas as pl


def kernel(x, edge_index):
    raise NotImplementedError("write your pallas kernel here")


# speedup vs baseline: 41.0458x; 41.0458x over previous
import functools

import jax
import jax.numpy as jnp
from jax import lax
from jax.experimental import pallas as pl
from jax.experimental.pallas import tpu as pltpu
from jax.experimental.pallas import tpu_sc as plsc

N = 10000
E = 320000
D = 128
NC, NS, L = 2, 16, 16
NW = NC * NS
N_PAD = 10240
RPT = N_PAD // NS
C = 128
E2 = 2 * E
G = 32
CHUNKS = 160
EPW = CHUNKS * C
E_TOT = EPW * NW

_mesh = plsc.VectorSubcoreMesh(core_axis_name="c", subcore_axis_name="s")


@functools.partial(
    pl.kernel,
    out_type=jax.ShapeDtypeStruct((NC, N_PAD), jnp.float32),
    mesh=_mesh,
    scratch_types=[
        pltpu.VMEM((CHUNKS, C), jnp.int32),
        pltpu.VMEM((C,), jnp.float32),
        pltpu.VMEM((RPT,), jnp.float32),
        pltpu.VMEM_SHARED((N_PAD,), jnp.float32),
    ],
)
def _degree_kernel(srcp_hbm, hist_hbm, idx_v, ones_v, stage_v, hist_sh):
    cid = lax.axis_index("c")
    sid = lax.axis_index("s")
    wid = cid * NS + sid
    for k in range(C // L):
        ones_v[pl.ds(k * L, L)] = jnp.ones((L,), jnp.float32)
    for k in range(RPT // L):
        stage_v[pl.ds(k * L, L)] = jnp.zeros((L,), jnp.float32)
    pltpu.sync_copy(stage_v, hist_sh.at[pl.ds(sid * RPT, RPT)])
    plsc.subcore_barrier()
    pltpu.sync_copy(srcp_hbm.at[wid], idx_v)

    @pl.loop(0, CHUNKS)
    def _chunk(j):
        pltpu.sync_copy(ones_v, hist_sh.at[idx_v.at[j]], add=True)

    plsc.subcore_barrier()
    pltpu.sync_copy(hist_sh.at[pl.ds(sid * RPT, RPT)], stage_v)
    pltpu.sync_copy(stage_v, hist_hbm.at[cid, pl.ds(sid * RPT, RPT)])


def _isd_body(h_ref, isd_ref):
    deg = h_ref[0] + h_ref[1]
    isd_ref[...] = lax.rsqrt(jnp.maximum(deg, 1.0))


def _scale_body(x_ref, c_ref, y_ref):
    y_ref[...] = x_ref[...] * c_ref[...]


@functools.partial(
    pl.kernel,
    out_type=jax.ShapeDtypeStruct((NC, N_PAD, D), jnp.float32),
    mesh=_mesh,
    scratch_types=[
        pltpu.VMEM((G, C), jnp.int32),
        pltpu.VMEM((G, C), jnp.int32),
        pltpu.VMEM((C, D), jnp.float32),
        pltpu.SemaphoreType.DMA,
        pltpu.SemaphoreType.DMA,
        pltpu.VMEM_SHARED((N_PAD, D), jnp.float32),
    ],
)
def _spmm_kernel(y_hbm, srcp_hbm, dstp_hbm, z_hbm, out_hbm,
                 src_v, dst_v, rows_v, gsem, ssem, acc):
    cid = lax.axis_index("c")
    sid = lax.axis_index("s")
    wid = cid * NS + sid
    base = sid * RPT
    for j in range(RPT // C):
        pltpu.sync_copy(z_hbm, acc.at[pl.ds(base + j * C, C)])
    plsc.subcore_barrier()

    @pl.loop(0, CHUNKS // G)
    def _group(g):
        pltpu.sync_copy(srcp_hbm.at[wid, pl.ds(g * G, G)], src_v)
        pltpu.sync_copy(dstp_hbm.at[wid, pl.ds(g * G, G)], dst_v)

        @pl.loop(0, G)
        def _edge_chunk(j):
            pltpu.async_copy(y_hbm.at[dst_v.at[j]], rows_v, gsem).wait()
            pltpu.async_copy(rows_v, acc.at[src_v.at[j]], ssem, add=True).wait()

    plsc.subcore_barrier()
    for j in range(RPT // C):
        pltpu.sync_copy(acc.at[pl.ds(base + j * C, C)], rows_v)
        pltpu.sync_copy(rows_v, out_hbm.at[cid, pl.ds(base + j * C, C)])


def _combine_body(p0_ref, p1_ref, c_ref, o_ref):
    o_ref[...] = (p0_ref[...] + p1_ref[...]) * c_ref[...]


def kernel(x, edge_index):
    e0 = edge_index[0]
    e1 = edge_index[1]
    src = jnp.concatenate([e0, e1])
    dst = jnp.concatenate([e1, e0])
    npad = E_TOT - E2
    pad_idx = (N + (jnp.arange(npad, dtype=jnp.int32) % (N_PAD - N)))
    srcp = jnp.concatenate([src, pad_idx]).reshape(NW, CHUNKS, C)
    dstp = jnp.concatenate([dst, pad_idx]).reshape(NW, CHUNKS, C)

    hist = _degree_kernel(srcp)

    isd = pl.pallas_call(
        _isd_body,
        out_shape=jax.ShapeDtypeStruct((N_PAD // D, D), jnp.float32),
    )(hist.reshape(NC, N_PAD // D, D))
    isd_col = isd.reshape(N_PAD)[:N, None]

    y = pl.pallas_call(
        _scale_body,
        out_shape=jax.ShapeDtypeStruct((N, D), jnp.float32),
    )(x, isd_col)

    y_pad = jnp.concatenate([y, jnp.zeros((N_PAD - N, D), jnp.float32)])
    zrows = jnp.zeros((C, D), jnp.float32)
    part = _spmm_kernel(y_pad, srcp, dstp, zrows)

    out = pl.pallas_call(
        _combine_body,
        out_shape=jax.ShapeDtypeStruct((N, D), jnp.float32),
    )(part[0, :N], part[1, :N], isd_col)
    return out


# speedup vs baseline: 64.2098x; 1.5643x over previous
import functools

import jax
import jax.numpy as jnp
from jax import lax
from jax.experimental import pallas as pl
from jax.experimental.pallas import tpu as pltpu
from jax.experimental.pallas import tpu_sc as plsc

N = 10000
E = 320000
D = 128
NC, NS, L = 2, 16, 16
NW = NC * NS
N_PAD = 10240
RPT = N_PAD // NS
C = 125
CW = E // (NW * C)
G = 16
NG = CW // G
FCW = 2 * E // (NW * C)

_mesh = plsc.VectorSubcoreMesh(core_axis_name="c", subcore_axis_name="s")


@functools.partial(
    pl.kernel,
    out_type=jax.ShapeDtypeStruct((NC, N_PAD), jnp.float32),
    mesh=_mesh,
    scratch_types=[
        pltpu.VMEM((FCW, C), jnp.int32),
        pltpu.VMEM((128,), jnp.float32),
        pltpu.VMEM((RPT,), jnp.float32),
        pltpu.VMEM_SHARED((N_PAD,), jnp.float32),
    ],
)
def _degree_kernel(idx_hbm, hist_hbm, idx_v, ones_v, stage_v, hist_sh):
    cid = lax.axis_index("c")
    sid = lax.axis_index("s")
    wid = cid * NS + sid
    for k in range(128 // L):
        ones_v[pl.ds(k * L, L)] = jnp.ones((L,), jnp.float32)
    for k in range(RPT // L):
        stage_v[pl.ds(k * L, L)] = jnp.zeros((L,), jnp.float32)
    pltpu.sync_copy(stage_v, hist_sh.at[pl.ds(sid * RPT, RPT)])
    plsc.subcore_barrier()
    pltpu.sync_copy(idx_hbm.at[wid], idx_v)

    @pl.loop(0, FCW)
    def _chunk(j):
        pltpu.sync_copy(ones_v.at[pl.ds(0, C)], hist_sh.at[idx_v.at[j]],
                        add=True)

    plsc.subcore_barrier()
    pltpu.sync_copy(hist_sh.at[pl.ds(sid * RPT, RPT)], stage_v)
    pltpu.sync_copy(stage_v, hist_hbm.at[cid, pl.ds(sid * RPT, RPT)])


def _isd_body(h_ref, isd_ref):
    deg = h_ref[0] + h_ref[1]
    isd_ref[...] = lax.rsqrt(jnp.maximum(deg, 1.0))


def _scale_body(x_ref, c_ref, y_ref):
    y_ref[...] = x_ref[...] * c_ref[...]


@functools.partial(
    pl.kernel,
    out_type=jax.ShapeDtypeStruct((NC, N_PAD, D), jnp.float32),
    mesh=_mesh,
    scratch_types=[
        pltpu.VMEM((G, C), jnp.int32),
        pltpu.VMEM((G, C), jnp.int32),
        pltpu.VMEM((128, D), jnp.float32),
        pltpu.VMEM((128, D), jnp.float32),
        pltpu.SemaphoreType.DMA,
        pltpu.SemaphoreType.DMA,
        pltpu.SemaphoreType.DMA,
        pltpu.SemaphoreType.DMA,
        pltpu.VMEM_SHARED((N_PAD, D), jnp.float32),
    ],
)
def _spmm_kernel(y_hbm, ei_hbm, z_hbm, out_hbm,
                 e0g, e1g, r0, r1, gs0, gs1, ss0, ss1, acc):
    cid = lax.axis_index("c")
    sid = lax.axis_index("s")
    wid = cid * NS + sid
    base = sid * RPT
    for j in range(RPT // 128):
        pltpu.sync_copy(z_hbm, acc.at[pl.ds(base + j * 128, 128)])
    plsc.subcore_barrier()

    r0s = r0.at[pl.ds(0, C)]
    r1s = r1.at[pl.ds(0, C)]

    @pl.loop(0, NG)
    def _group(g):
        pltpu.sync_copy(ei_hbm.at[0, wid, pl.ds(g * G, G)], e0g)
        pltpu.sync_copy(ei_hbm.at[1, wid, pl.ds(g * G, G)], e1g)
        pltpu.async_copy(y_hbm.at[e1g.at[0]], r0s, gs0)

        @pl.loop(0, G)
        def _chunk(h):
            pltpu.async_copy(y_hbm.at[e0g.at[h]], r1s, gs1)
            pltpu.make_async_copy(y_hbm.at[e1g.at[h]], r0s, gs0).wait()
            sd0 = pltpu.async_copy(r0s, acc.at[e0g.at[h]], ss0, add=True)
            sd0.wait()

            @pl.when(h < G - 1)
            def _():
                pltpu.async_copy(y_hbm.at[e1g.at[h + 1]], r0s, gs0)

            pltpu.make_async_copy(y_hbm.at[e0g.at[h]], r1s, gs1).wait()
            sd1 = pltpu.async_copy(r1s, acc.at[e1g.at[h]], ss1, add=True)
            sd1.wait()

    plsc.subcore_barrier()
    for j in range(RPT // 128):
        pltpu.sync_copy(acc.at[pl.ds(base + j * 128, 128)], r0)
        pltpu.sync_copy(r0, out_hbm.at[cid, pl.ds(base + j * 128, 128)])


def _combine_body(p_ref, c_ref, o_ref):
    o_ref[...] = (p_ref[0, :N, :] + p_ref[1, :N, :]) * c_ref[...]


def kernel(x, edge_index):
    idx_flat = edge_index.reshape(NW, FCW, C)
    ei = edge_index.reshape(2, NW, CW, C)

    hist = _degree_kernel(idx_flat)

    isd = pl.pallas_call(
        _isd_body,
        out_shape=jax.ShapeDtypeStruct((N_PAD // D, D), jnp.float32),
    )(hist.reshape(NC, N_PAD // D, D))
    isd_col = isd.reshape(N_PAD)[:N, None]

    y = pl.pallas_call(
        _scale_body,
        out_shape=jax.ShapeDtypeStruct((N, D), jnp.float32),
    )(x, isd_col)

    zrows = jnp.zeros((128, D), jnp.float32)
    part = _spmm_kernel(y, ei, zrows)

    out = pl.pallas_call(
        _combine_body,
        out_shape=jax.ShapeDtypeStruct((N, D), jnp.float32),
    )(part, isd_col)
    return out
